# Initial kernel scaffold; baseline (speedup 1.0000x reference)
import jax
import jax.numpy as jnp
from jax.experimental import pallas as pl


def kernel(node_feats, W_node, W_edge, cob_node, cob_edge, edge_index):
    raise NotImplementedError("write your pallas kernel here")


# speedup vs baseline: 5.6938x; 5.6938x over previous
import functools

import jax
import jax.numpy as jnp
from jax import lax
from jax.experimental import pallas as pl
from jax.experimental.pallas import tpu as pltpu
from jax.experimental.pallas import tpu_sc as plsc

N = 10000
E = 320000
D = 128
IRR = 25
BW = 25
TW = 32

_NC = 2
_NS = 16
_NW = _NC * _NS
_EW = E // _NW
_IW = 125
_CH = 1000
_CR = _CH // _IW
_G = _EW // _CH
_NCHUNK = 624
_NODE_WORKERS = 16
_NTAIL = N - _NCHUNK * _NODE_WORKERS


def _tc_body(x_ref, wn_ref, we_ref, cobn_ref, cobe_ref,
             node_ref, p_ref, q_ref):
    cobn = cobn_ref[...]
    cobe = cobe_ref[...]
    we = we_ref[...]
    m = jnp.dot(wn_ref[...], cobn, preferred_element_type=jnp.float32)
    a = jnp.dot(we[:D, :], cobe, preferred_element_type=jnp.float32)
    b = jnp.dot(we[D:, :], cobe, preferred_element_type=jnp.float32)
    x = x_ref[...]
    node_ref[...] = jnp.dot(x, m, preferred_element_type=jnp.float32)
    p_ref[...] = jnp.dot(x, a, preferred_element_type=jnp.float32)
    q_ref[...] = jnp.dot(x, b, preferred_element_type=jnp.float32)


_ROWS_PER_BLK = 1000

_tc_matmul = pl.pallas_call(
    _tc_body,
    grid=(N // _ROWS_PER_BLK,),
    in_specs=[
        pl.BlockSpec((_ROWS_PER_BLK, D), lambda i: (i, 0)),
        pl.BlockSpec((D, IRR), lambda i: (0, 0)),
        pl.BlockSpec((2 * D, IRR), lambda i: (0, 0)),
        pl.BlockSpec((IRR, BW), lambda i: (0, 0)),
        pl.BlockSpec((IRR, TW), lambda i: (0, 0)),
    ],
    out_specs=[
        pl.BlockSpec((_ROWS_PER_BLK, BW), lambda i: (i, 0)),
        pl.BlockSpec((_ROWS_PER_BLK, TW), lambda i: (i, 0)),
        pl.BlockSpec((_ROWS_PER_BLK, TW), lambda i: (i, 0)),
    ],
    out_shape=[
        jax.ShapeDtypeStruct((N, BW), jnp.float32),
        jax.ShapeDtypeStruct((N, TW), jnp.float32),
        jax.ShapeDtypeStruct((N, TW), jnp.float32),
    ],
)


_sc_mesh = plsc.VectorSubcoreMesh(core_axis_name="c", subcore_axis_name="s")


@functools.partial(
    pl.kernel,
    mesh=_sc_mesh,
    out_type=jax.ShapeDtypeStruct((N + E, BW), jnp.float32),
    compiler_params=pltpu.CompilerParams(use_tc_tiling_on_sc=False),
    scratch_types=[
        pltpu.VMEM((_CR, _IW), jnp.int32),
        pltpu.VMEM((_CR, _IW), jnp.int32),
        pltpu.VMEM((_CH, TW), jnp.float32),
        pltpu.VMEM((_CH, TW), jnp.float32),
        pltpu.VMEM((_CH, BW), jnp.float32),
        pltpu.VMEM((_NCHUNK, BW), jnp.float32),
        pltpu.SemaphoreType.DMA,
    ],
)
def _sc_edge(node_hbm, p_hbm, q_hbm, src_hbm, dst_hbm, out_hbm,
             src_v, dst_v, rows_p, rows_q, out_v, node_v, sem):
    wid = lax.axis_index("s") * _NC + lax.axis_index("c")

    @pl.when(wid < _NODE_WORKERS)
    def _():
        nb = wid * _NCHUNK
        pltpu.sync_copy(node_hbm.at[pl.ds(nb, _NCHUNK)], node_v)
        pltpu.sync_copy(node_v, out_hbm.at[pl.ds(nb, _NCHUNK)])

    @pl.when(wid == _NODE_WORKERS)
    def _():
        nb = _NODE_WORKERS * _NCHUNK
        tail = node_v.at[pl.ds(0, _NTAIL)]
        pltpu.sync_copy(node_hbm.at[pl.ds(nb, _NTAIL)], tail)
        pltpu.sync_copy(tail, out_hbm.at[pl.ds(nb, _NTAIL)])

    row0 = wid * (_EW // _IW)

    def chunk_body(g, _):
        r0 = row0 + g * _CR
        pltpu.sync_copy(src_hbm.at[pl.ds(r0, _CR)], src_v)
        pltpu.sync_copy(dst_hbm.at[pl.ds(r0, _CR)], dst_v)
        copies = []
        for j in range(_CR):
            copies.append(pltpu.async_copy(
                p_hbm.at[src_v.at[j]], rows_p.at[pl.ds(j * _IW, _IW)], sem))
            copies.append(pltpu.async_copy(
                q_hbm.at[dst_v.at[j]], rows_q.at[pl.ds(j * _IW, _IW)], sem))
        for c in copies:
            c.wait()

        def add_body(r8, _):
            r = r8 * 8
            for u in range(8):
                lo = rows_p[r + u, pl.ds(0, 16)] + rows_q[r + u, pl.ds(0, 16)]
                hi = (rows_p[r + u, pl.ds(BW - 16, 16)]
                      + rows_q[r + u, pl.ds(BW - 16, 16)])
                out_v[r + u, pl.ds(0, 16)] = lo
                out_v[r + u, pl.ds(BW - 16, 16)] = hi
            return 0

        lax.fori_loop(0, _CH // 8, add_body, 0)
        e0 = wid * _EW + g * _CH
        pltpu.sync_copy(out_v, out_hbm.at[pl.ds(N + e0, _CH)])
        return 0

    lax.fori_loop(0, _G, chunk_body, 0)


def kernel(node_feats, W_node, W_edge, cob_node, cob_edge, edge_index):
    cobn = cob_node.reshape(IRR, BW)
    cobe = cob_edge.reshape(IRR, BW)
    cobe_pad = jnp.zeros((IRR, TW), jnp.float32).at[:, :BW].set(cobe)
    node_out, p32, q32 = _tc_matmul(node_feats, W_node, W_edge, cobn, cobe_pad)
    src2d = edge_index[0].reshape(E // _IW, _IW)
    dst2d = edge_index[1].reshape(E // _IW, _IW)
    return _sc_edge(node_out, p32, q32, src2d, dst2d)
